# baseline (speedup 1.0000x reference)
import jax
import jax.numpy as jnp
from jax import lax
from jax.experimental import pallas as pl
from jax.experimental.pallas import tpu as pltpu
from jax.experimental.pallas import tpu_sc as plsc

_D = 32
_NC, _NS = 2, 16
_NW = _NC * _NS
_CHUNK = 128
_NFIRE = 8
_SUPER = _CHUNK * _NFIRE
_NT_FULL = 6254
_TAIL = 20


def _relayout_body(tab_hbm, tail_hbm, out_hbm,
                   slab0, slab1, buf0, buf1, tbuf,
                   gs0, gs1, ws0, ws1):
    wid = lax.axis_index("s") * _NC + lax.axis_index("c")
    lo = jnp.minimum(wid * 196, _NT_FULL)
    hi = jnp.minimum(lo + 196, _NT_FULL)
    iota32 = lax.iota(jnp.int32, 16) * _D

    def _flip(slab, buf):
        for grp in range(8):
            for g in range(4):
                for ds in range(8):
                    val = slab[g, ds, pl.ds(grp * 16, 16)]
                    plsc.store_scatter(
                        buf, [iota32 + (grp * 16 * _D + g * 8 + ds)], val)

    def _pair(j, carry):
        tc0 = lo + 2 * j
        tc1 = tc0 + 1
        for g in range(4):
            pltpu.async_copy(tab_hbm.at[g, :, pl.ds(tc0 * 128, 128)],
                             slab0.at[g], gs0)
        for g in range(4):
            pltpu.async_copy(tab_hbm.at[g, :, pl.ds(tc1 * 128, 128)],
                             slab1.at[g], gs1)
        for g in range(4):
            pltpu.make_async_copy(tab_hbm.at[0, :, pl.ds(0, 128)],
                                  slab0.at[g], gs0).wait()

        @pl.when(j > 0)
        def _():
            pltpu.make_async_copy(buf0, out_hbm.at[pl.ds(0, 4096)], ws0).wait()
        _flip(slab0, buf0)
        pltpu.async_copy(buf0, out_hbm.at[pl.ds(tc0 * 4096, 4096)], ws0)
        for g in range(4):
            pltpu.make_async_copy(tab_hbm.at[0, :, pl.ds(0, 128)],
                                  slab1.at[g], gs1).wait()

        @pl.when(j > 0)
        def _():
            pltpu.make_async_copy(buf1, out_hbm.at[pl.ds(0, 4096)], ws1).wait()
        _flip(slab1, buf1)
        pltpu.async_copy(buf1, out_hbm.at[pl.ds(tc1 * 4096, 4096)], ws1)
        return carry

    lax.fori_loop(0, (hi - lo) // 2, _pair, 0)
    pltpu.make_async_copy(buf0, out_hbm.at[pl.ds(0, 4096)], ws0).wait()
    pltpu.make_async_copy(buf1, out_hbm.at[pl.ds(0, 4096)], ws1).wait()

    @pl.when(wid == _NW - 1)
    def _():
        pltpu.sync_copy(tail_hbm, tbuf)
        pltpu.sync_copy(tbuf, out_hbm.at[pl.ds(_NT_FULL * 4096, _TAIL * _D)])
        return


def _gather_body(ids_hbm, offs_hbm, tab_hbm, out_hbm,
                 idx_v, offs_v, buf0, buf1, gsem0, gsem1):
    wid = lax.axis_index("s") * _NC + lax.axis_index("c")
    n_chunks = ids_hbm.shape[1]
    n_super = n_chunks // _NFIRE
    rows_per_w = n_chunks * _CHUNK
    base = wid * rows_per_w

    pltpu.sync_copy(ids_hbm.at[wid], idx_v)
    pltpu.sync_copy(offs_hbm, offs_v)
    off = offs_v[...]

    def _add_off(j, carry):
        for k in range(_CHUNK // 16):
            sl = (j, pl.ds(k * 16, 16))
            idx_v[sl] = idx_v[sl] + off
        return carry
    lax.fori_loop(0, n_chunks, _add_off, 0)

    def _fire(s, buf, sem):
        for k in range(_NFIRE):
            pltpu.async_copy(
                tab_hbm.at[idx_v.at[s * _NFIRE + k]],
                buf.at[pl.ds(k * _CHUNK, _CHUNK)],
                sem)

    def _drain(buf, sem):
        for k in range(_NFIRE):
            pltpu.make_async_copy(
                tab_hbm.at[idx_v.at[0]],
                buf.at[pl.ds(k * _CHUNK, _CHUNK)],
                sem).wait()

    def _super_pair(s2, carry):
        s = s2 * 2
        _fire(s, buf0, gsem0)
        _drain(buf0, gsem0)
        _fire(s + 1, buf1, gsem1)
        pltpu.sync_copy(buf0, out_hbm.at[pl.ds(base + s * _SUPER, _SUPER)])
        _drain(buf1, gsem1)
        pltpu.sync_copy(buf1, out_hbm.at[pl.ds(base + (s + 1) * _SUPER, _SUPER)])
        return carry
    lax.fori_loop(0, n_super // 2, _super_pair, 0)


def kernel(input_ids, embedding, offsets):
    B, T, H = input_ids.shape
    R = B * T * H
    V = embedding.shape[0]
    rows_per_w = R // _NW
    n_chunks = rows_per_w // _CHUNK
    Vp = (_NT_FULL + 1) * 128

    mesh = plsc.VectorSubcoreMesh(core_axis_name="c", subcore_axis_name="s",
                                  num_cores=_NC, num_subcores=_NS)

    tab_native = embedding.T.reshape(4, 8, V)
    tail = embedding[_NT_FULL * 128:, :].reshape(-1)
    relayout = pl.kernel(
        _relayout_body,
        out_type=jax.ShapeDtypeStruct((Vp * _D,), jnp.float32),
        mesh=mesh,
        scratch_types=[
            pltpu.VMEM((4, 8, 128), jnp.float32),
            pltpu.VMEM((4, 8, 128), jnp.float32),
            pltpu.VMEM((4096,), jnp.float32),
            pltpu.VMEM((4096,), jnp.float32),
            pltpu.VMEM((_TAIL * _D,), jnp.float32),
        ] + [pltpu.SemaphoreType.DMA] * 4,
        compiler_params=pltpu.CompilerParams(use_tc_tiling_on_sc=True,
                                             needs_layout_passes=False),
    )
    tab_lin = relayout(tab_native, tail).reshape(Vp, _D)

    ids_flat = input_ids.reshape(_NW, n_chunks, _CHUNK).astype(jnp.int32)
    offs16 = jnp.tile(offsets.astype(jnp.int32), 16 // H)
    run = pl.kernel(
        _gather_body,
        out_type=jax.ShapeDtypeStruct((R, _D), jnp.float32),
        mesh=mesh,
        scratch_types=[
            pltpu.VMEM((n_chunks, _CHUNK), jnp.int32),
            pltpu.VMEM((16,), jnp.int32),
            pltpu.VMEM((_SUPER, _D), jnp.float32),
            pltpu.VMEM((_SUPER, _D), jnp.float32),
            pltpu.SemaphoreType.DMA,
            pltpu.SemaphoreType.DMA,
        ],
        compiler_params=pltpu.CompilerParams(use_tc_tiling_on_sc=False),
    )
    out = run(ids_flat, offs16, tab_lin)
    return out.reshape(B, T, H, _D)


# speedup vs baseline: 1.0243x; 1.0243x over previous
import jax
import jax.numpy as jnp
from jax import lax
from jax.experimental import pallas as pl
from jax.experimental.pallas import tpu as pltpu
from jax.experimental.pallas import tpu_sc as plsc

_D = 32
_NC, _NS = 2, 16
_NW = _NC * _NS
_CHUNK = 128
_NFIRE = 8
_SUPER = _CHUNK * _NFIRE
_NT_FULL = 6254
_TAIL = 20
_KCOL = 6


def _relayout_body(tab_hbm, tail_hbm, out_hbm,
                   slab0, slab1, buf0, buf1, tbuf,
                   gs0, gs1, ws0, ws1):
    wid = lax.axis_index("s") * _NC + lax.axis_index("c")
    n_groups = (_NT_FULL + _KCOL - 1) // _KCOL
    per_w = (n_groups + _NW - 1) // _NW
    lo_g = wid * per_w
    n = jnp.maximum(0, jnp.minimum(per_w, n_groups - lo_g))
    smax = (_NT_FULL - _KCOL) * 128
    iota32 = lax.iota(jnp.int32, 16) * _D
    kw = _KCOL * 128

    def _flip(slab, buf):
        def _k(k, c):
            kb = k * (128 * _D)
            for grp in range(8):
                for g in range(4):
                    for ds in range(8):
                        val = slab[g, ds, pl.ds(k * 128 + grp * 16, 16)]
                        plsc.store_scatter(
                            buf,
                            [iota32 + (kb + grp * 16 * _D + g * 8 + ds)],
                            val)
            return c
        lax.fori_loop(0, _KCOL, _k, 0)

    def _half(j, g_idx, slab, buf, gs, ws):
        s = jnp.minimum(g_idx * (_KCOL * 128), smax)
        for g in range(4):
            pltpu.make_async_copy(tab_hbm.at[0, :, pl.ds(0, kw)],
                                  slab.at[g], gs).wait()

        @pl.when(j > 0)
        def _():
            pltpu.make_async_copy(buf, out_hbm.at[pl.ds(0, kw * _D)],
                                  ws).wait()
        _flip(slab, buf)
        pltpu.async_copy(buf, out_hbm.at[pl.ds(s * _D, kw * _D)], ws)

    def _fire_in(g_idx, slab, gs):
        s = jnp.minimum(g_idx * (_KCOL * 128), smax)
        for g in range(4):
            pltpu.async_copy(tab_hbm.at[g, :, pl.ds(s, kw)], slab.at[g], gs)

    def _pair(j, carry):
        g0 = lo_g + 2 * j
        g1 = g0 + 1
        _fire_in(g0, slab0, gs0)
        _fire_in(g1, slab1, gs1)
        _half(j, g0, slab0, buf0, gs0, ws0)
        _half(j, g1, slab1, buf1, gs1, ws1)
        return carry

    lax.fori_loop(0, (n + 1) // 2, _pair, 0)
    pltpu.make_async_copy(buf0, out_hbm.at[pl.ds(0, kw * _D)], ws0).wait()
    pltpu.make_async_copy(buf1, out_hbm.at[pl.ds(0, kw * _D)], ws1).wait()

    @pl.when(wid == _NW - 1)
    def _():
        pltpu.sync_copy(tail_hbm, tbuf)
        pltpu.sync_copy(tbuf, out_hbm.at[pl.ds(_NT_FULL * 4096, _TAIL * _D)])
        return


def _gather_body(ids_hbm, offs_hbm, tab_hbm, out_hbm,
                 idx_v, offs_v, buf0, buf1, gsem0, gsem1):
    wid = lax.axis_index("s") * _NC + lax.axis_index("c")
    n_chunks = ids_hbm.shape[1]
    n_super = n_chunks // _NFIRE
    rows_per_w = n_chunks * _CHUNK
    base = wid * rows_per_w

    pltpu.sync_copy(ids_hbm.at[wid], idx_v)
    pltpu.sync_copy(offs_hbm, offs_v)
    off = offs_v[...]

    def _add_off(j, carry):
        for k in range(_CHUNK // 16):
            sl = (j, pl.ds(k * 16, 16))
            idx_v[sl] = idx_v[sl] + off
        return carry
    lax.fori_loop(0, n_chunks, _add_off, 0)

    def _fire(s, buf, sem):
        for k in range(_NFIRE):
            pltpu.async_copy(
                tab_hbm.at[idx_v.at[s * _NFIRE + k]],
                buf.at[pl.ds(k * _CHUNK, _CHUNK)],
                sem)

    def _drain(buf, sem):
        for k in range(_NFIRE):
            pltpu.make_async_copy(
                tab_hbm.at[idx_v.at[0]],
                buf.at[pl.ds(k * _CHUNK, _CHUNK)],
                sem).wait()

    def _super_pair(s2, carry):
        s = s2 * 2
        _fire(s, buf0, gsem0)
        _drain(buf0, gsem0)
        _fire(s + 1, buf1, gsem1)
        pltpu.sync_copy(buf0, out_hbm.at[pl.ds(base + s * _SUPER, _SUPER)])
        _drain(buf1, gsem1)
        pltpu.sync_copy(buf1, out_hbm.at[pl.ds(base + (s + 1) * _SUPER, _SUPER)])
        return carry
    lax.fori_loop(0, n_super // 2, _super_pair, 0)


def kernel(input_ids, embedding, offsets):
    B, T, H = input_ids.shape
    R = B * T * H
    V = embedding.shape[0]
    rows_per_w = R // _NW
    n_chunks = rows_per_w // _CHUNK
    Vp = (_NT_FULL + 1) * 128

    mesh = plsc.VectorSubcoreMesh(core_axis_name="c", subcore_axis_name="s",
                                  num_cores=_NC, num_subcores=_NS)

    tab_native = embedding.T.reshape(4, 8, V)
    tail = embedding[_NT_FULL * 128:, :].reshape(-1)
    relayout = pl.kernel(
        _relayout_body,
        out_type=jax.ShapeDtypeStruct((Vp * _D,), jnp.float32),
        mesh=mesh,
        scratch_types=[
            pltpu.VMEM((4, 8, _KCOL * 128), jnp.float32),
            pltpu.VMEM((4, 8, _KCOL * 128), jnp.float32),
            pltpu.VMEM((_KCOL * 128 * _D,), jnp.float32),
            pltpu.VMEM((_KCOL * 128 * _D,), jnp.float32),
            pltpu.VMEM((_TAIL * _D,), jnp.float32),
        ] + [pltpu.SemaphoreType.DMA] * 4,
        compiler_params=pltpu.CompilerParams(use_tc_tiling_on_sc=True,
                                             needs_layout_passes=False),
    )
    tab_lin = relayout(tab_native, tail).reshape(Vp, _D)

    ids_flat = input_ids.reshape(_NW, n_chunks, _CHUNK).astype(jnp.int32)
    offs16 = jnp.tile(offsets.astype(jnp.int32), 16 // H)
    run = pl.kernel(
        _gather_body,
        out_type=jax.ShapeDtypeStruct((R, _D), jnp.float32),
        mesh=mesh,
        scratch_types=[
            pltpu.VMEM((n_chunks, _CHUNK), jnp.int32),
            pltpu.VMEM((16,), jnp.int32),
            pltpu.VMEM((_SUPER, _D), jnp.float32),
            pltpu.VMEM((_SUPER, _D), jnp.float32),
            pltpu.SemaphoreType.DMA,
            pltpu.SemaphoreType.DMA,
        ],
        compiler_params=pltpu.CompilerParams(use_tc_tiling_on_sc=False),
    )
    out = run(ids_flat, offs16, tab_lin)
    return out.reshape(B, T, H, _D)


# speedup vs baseline: 1.3680x; 1.3356x over previous
import functools

import jax
import jax.numpy as jnp
from jax import lax
from jax.experimental import pallas as pl
from jax.experimental.pallas import tpu as pltpu
from jax.experimental.pallas import tpu_sc as plsc

_D = 32
_NC, _NS = 2, 16
_NW = _NC * _NS
_CHUNK = 128
_NFIRE = 8
_SUPER = _CHUNK * _NFIRE


def _gather_body(ids_hbm, offs_hbm, table_hbm, out_hbm,
                 idx_v, offs_v, buf0, buf1, gsem0, gsem1):
    wid = lax.axis_index("s") * _NC + lax.axis_index("c")
    n_chunks = ids_hbm.shape[1]
    n_super = n_chunks // _NFIRE
    rows_per_w = n_chunks * _CHUNK
    base = wid * rows_per_w

    pltpu.sync_copy(ids_hbm.at[wid], idx_v)
    pltpu.sync_copy(offs_hbm, offs_v)
    off = offs_v[...]

    def _add_off(j, carry):
        for k in range(_CHUNK // 16):
            sl = (j, pl.ds(k * 16, 16))
            idx_v[sl] = idx_v[sl] + off
        return carry
    lax.fori_loop(0, n_chunks, _add_off, 0)

    def _fire(s, buf, sem):
        for k in range(_NFIRE):
            pltpu.async_copy(
                table_hbm.at[idx_v.at[s * _NFIRE + k]],
                buf.at[pl.ds(k * _CHUNK, _CHUNK)],
                sem)

    def _drain(buf, sem):
        for k in range(_NFIRE):
            pltpu.make_async_copy(
                table_hbm.at[idx_v.at[0]],
                buf.at[pl.ds(k * _CHUNK, _CHUNK)],
                sem).wait()

    def _super_pair(s2, carry):
        s = s2 * 2
        _fire(s, buf0, gsem0)
        _drain(buf0, gsem0)
        _fire(s + 1, buf1, gsem1)
        pltpu.sync_copy(buf0, out_hbm.at[pl.ds(base + s * _SUPER, _SUPER)])
        _drain(buf1, gsem1)
        pltpu.sync_copy(buf1, out_hbm.at[pl.ds(base + (s + 1) * _SUPER, _SUPER)])
        return carry
    lax.fori_loop(0, n_super // 2, _super_pair, 0)


def kernel(input_ids, embedding, offsets):
    B, T, H = input_ids.shape
    R = B * T * H
    rows_per_w = R // _NW
    n_chunks = rows_per_w // _CHUNK

    ids_flat = input_ids.reshape(_NW, n_chunks, _CHUNK).astype(jnp.int32)
    offs16 = jnp.tile(offsets.astype(jnp.int32), 16 // H)

    mesh = plsc.VectorSubcoreMesh(core_axis_name="c", subcore_axis_name="s",
                                  num_cores=_NC, num_subcores=_NS)
    run = pl.kernel(
        _gather_body,
        out_type=jax.ShapeDtypeStruct((R, _D), jnp.float32),
        mesh=mesh,
        scratch_types=[
            pltpu.VMEM((n_chunks, _CHUNK), jnp.int32),
            pltpu.VMEM((16,), jnp.int32),
            pltpu.VMEM((_SUPER, _D), jnp.float32),
            pltpu.VMEM((_SUPER, _D), jnp.float32),
            pltpu.SemaphoreType.DMA,
            pltpu.SemaphoreType.DMA,
        ],
        compiler_params=pltpu.CompilerParams(use_tc_tiling_on_sc=False),
    )
    out = run(ids_flat, offs16, embedding)
    return out.reshape(B, T, H, _D)
